# baseline (speedup 1.0000x reference)
import functools

import jax
import jax.numpy as jnp
from jax import lax
from jax.experimental import pallas as pl
from jax.experimental.pallas import tpu as pltpu
from jax.experimental.pallas import tpu_sc as plsc

N = 32768
D = 512
LANES = 16
NC = 2
NS = 16
NW = NC * NS

N_SC = N
N_TC = N - N_SC

RPW = N_SC // NW
CH = 32
NBUF = 4
NCHUNK = RPW // CH
KV = D // LANES
ROWU = 4

BR = 256
NB = N_TC // BR

_mesh = plsc.VectorSubcoreMesh(core_axis_name="c", subcore_axis_name="s")


@functools.partial(
    pl.kernel,
    out_type=(
        jax.ShapeDtypeStruct((NW, LANES), jnp.float32),
        jax.ShapeDtypeStruct((NW, LANES), jnp.int32),
    ),
    mesh=_mesh,
    compiler_params=pltpu.CompilerParams(needs_layout_passes=False),
    scratch_types=(
        pltpu.VMEM((2, D), jnp.float32),
        pltpu.VMEM((D,), jnp.float32),
        pltpu.VMEM((NBUF, CH, D), jnp.float32),
        pltpu.VMEM((1, LANES), jnp.float32),
        pltpu.VMEM((1, LANES), jnp.int32),
        (pltpu.SemaphoreType.DMA,) * NBUF,
    ),
)
def _scan_kernel(feats_hbm, w_hbm, vals_out, idx_out, w_v, wd_v, buf, rec_v,
                 rec_i, sems):
    ci = lax.axis_index("c")
    si = lax.axis_index("s")
    wid = si * NC + ci
    base = wid * RPW

    pltpu.sync_copy(w_hbm, w_v)
    for k in range(KV):
        wd_v[pl.ds(LANES * k, LANES)] = (
            w_v[0, pl.ds(LANES * k, LANES)]
            - w_v[1, pl.ds(LANES * k, LANES)])

    def start(cbase, slot):
        return pltpu.async_copy(
            feats_hbm.at[pl.ds(cbase, CH), :], buf.at[slot], sems[slot])

    for p in range(NBUF):
        start(base + p * CH, p)

    lane = lax.iota(jnp.int32, LANES)
    gmode = "promise_in_bounds"
    perms = {k: lane ^ k for k in (8, 4, 2, 1)}
    masks = {k: (lane & k) == 0 for k in (8, 4, 2, 1)}

    carry = (jnp.full((LANES,), -jnp.inf, jnp.float32),
             jnp.zeros((LANES,), jnp.int32),
             jnp.full((LANES,), jnp.inf, jnp.float32),
             jnp.zeros((LANES,), jnp.int32))

    def make_group_body(slot, cbase):
        def group_body(g, cr):
            vecs = []
            for j in range(LANES):
                r = g * LANES + j
                a = buf[slot, r, pl.ds(0, LANES)] * wd_v[pl.ds(0, LANES)]
                for k in range(1, KV):
                    a = a + (buf[slot, r, pl.ds(LANES * k, LANES)]
                             * wd_v[pl.ds(LANES * k, LANES)])
                vecs.append(a)
            for k in (8, 4, 2, 1):
                half = len(vecs) // 2
                nxt = []
                for i2 in range(half):
                    x, y = vecs[i2], vecs[i2 + half]
                    xa = x + x.at[perms[k]].get(mode=gmode)
                    yb = y + y.at[perms[k]].get(mode=gmode)
                    nxt.append(jnp.where(masks[k], xa, yb))
                vecs = nxt
            z = vecs[0]
            bmax16, bidx16, bmin16, midx16 = cr
            rows16 = (cbase + g * LANES) + lane
            upmax = z > bmax16
            bmax16 = jnp.where(upmax, z, bmax16)
            bidx16 = jnp.where(upmax, rows16, bidx16)
            upmin = z < bmin16
            bmin16 = jnp.where(upmin, z, bmin16)
            midx16 = jnp.where(upmin, rows16, midx16)
            return (bmax16, bidx16, bmin16, midx16)

        return group_body

    def chunk_body(g, cr):
        for b in range(NBUF):
            c = NBUF * g + b
            cbase = base + c * CH
            pltpu.make_async_copy(
                feats_hbm.at[pl.ds(cbase, CH), :], buf.at[b],
                sems[b]).wait()
            cr = lax.fori_loop(0, CH // LANES, make_group_body(b, cbase),
                               cr)

            @pl.when(c + NBUF < NCHUNK)
            def _():
                start(cbase + NBUF * CH, b)
        return cr

    carry = lax.fori_loop(0, NCHUNK // NBUF, chunk_body, carry)
    bmax16, bidx16, bmin16, midx16 = carry
    big = jnp.int32(2 ** 30)
    mv = jnp.max(bmax16)
    mi = jnp.min(jnp.where(bmax16 == mv, bidx16, big))
    nv = jnp.min(bmin16)
    ni = jnp.min(jnp.where(bmin16 == nv, midx16, big))
    rec_v[0] = jnp.where(lane == 0, mv,
                         jnp.where(lane == 1, nv,
                                   jnp.zeros((LANES,), jnp.float32)))
    rec_i[0] = jnp.where(lane == 0, mi,
                         jnp.where(lane == 1, ni,
                                   jnp.zeros((LANES,), jnp.int32)))
    pltpu.sync_copy(rec_v, vals_out.at[pl.ds(wid, 1)])
    pltpu.sync_copy(rec_i, idx_out.at[pl.ds(wid, 1)])


def _tc_scan_body(feats_blk, wdt_ref, vals_out, idx_out,
                  bmaxv, bmaxi, bminv, bmini):
    i = pl.program_id(0)

    @pl.when(i == 0)
    def _init():
        bmaxv[...] = jnp.full((BR, 1), -jnp.inf, jnp.float32)
        bmaxi[...] = jnp.zeros((BR, 1), jnp.int32)
        bminv[...] = jnp.full((BR, 1), jnp.inf, jnp.float32)
        bmini[...] = jnp.zeros((BR, 1), jnp.int32)

    x = feats_blk[...].reshape(BR, 4, 128)
    p = x * wdt_ref[...][None, :, :]
    d = jnp.sum(jnp.sum(p, axis=1), axis=1, keepdims=True)
    rows = (N_SC + i * BR
            + lax.broadcasted_iota(jnp.int32, (BR, 1), 0))
    upmax = d > bmaxv[...]
    bmaxv[...] = jnp.where(upmax, d, bmaxv[...])
    bmaxi[...] = jnp.where(upmax, rows, bmaxi[...])
    upmin = d < bminv[...]
    bminv[...] = jnp.where(upmin, d, bminv[...])
    bmini[...] = jnp.where(upmin, rows, bmini[...])

    @pl.when(i == NB - 1)
    def _finish():
        big = jnp.int32(2 ** 30)
        mv = jnp.max(bmaxv[...])
        mi = jnp.min(jnp.where(bmaxv[...] == mv, bmaxi[...], big))
        nv = jnp.min(bminv[...])
        ni = jnp.min(jnp.where(bminv[...] == nv, bmini[...], big))
        col = lax.broadcasted_iota(jnp.int32, (8, 128), 1)
        row8 = lax.broadcasted_iota(jnp.int32, (8, 128), 0)
        first = (col == 0) & (row8 == 0)
        second = (col == 1) & (row8 == 0)
        vals_out[...] = jnp.where(first, mv,
                                  jnp.where(second, nv,
                                            jnp.zeros((8, 128),
                                                      jnp.float32)))
        idx_out[...] = jnp.where(first, mi,
                                 jnp.where(second, ni,
                                           jnp.zeros((8, 128), jnp.int32)))


_tc_scan = None if N_TC == 0 else pl.pallas_call(
    _tc_scan_body,
    grid=(NB,),
    in_specs=[
        pl.BlockSpec((BR, D), lambda i: (i + N_SC // BR, 0)),
        pl.BlockSpec((4, 128), lambda i: (0, 0)),
    ],
    out_specs=[
        pl.BlockSpec((8, 128), lambda i: (0, 0)),
        pl.BlockSpec((8, 128), lambda i: (0, 0)),
    ],
    out_shape=[
        jax.ShapeDtypeStruct((8, 128), jnp.float32),
        jax.ShapeDtypeStruct((8, 128), jnp.int32),
    ],
    scratch_shapes=[
        pltpu.VMEM((BR, 1), jnp.float32),
        pltpu.VMEM((BR, 1), jnp.int32),
        pltpu.VMEM((BR, 1), jnp.float32),
        pltpu.VMEM((BR, 1), jnp.int32),
    ],
)


def _merge_body(cand_v, cand_i, tcv, tci, feats, out, sem0, sem1):
    bmaxv = cand_v[0, 0]
    bmaxi = cand_i[0, 0]
    bminv = cand_v[0, 1]
    bmini = cand_i[0, 1]
    for w in range(1, NW):
        v0 = cand_v[w, 0]
        i0 = cand_i[w, 0]
        t0 = (v0 > bmaxv) | ((v0 == bmaxv) & (i0 < bmaxi))
        bmaxv = jnp.where(t0, v0, bmaxv)
        bmaxi = jnp.where(t0, i0, bmaxi)
        v1 = cand_v[w, 1]
        i1 = cand_i[w, 1]
        t1 = (v1 < bminv) | ((v1 == bminv) & (i1 < bmini))
        bminv = jnp.where(t1, v1, bminv)
        bmini = jnp.where(t1, i1, bmini)
    if tcv is not None:
        v0 = tcv[0, 0]
        i0 = tci[0, 0]
        t0 = (v0 > bmaxv) | ((v0 == bmaxv) & (i0 < bmaxi))
        bmaxi = jnp.where(t0, i0, bmaxi)
        v1 = tcv[0, 1]
        i1 = tci[0, 1]
        t1 = (v1 < bminv) | ((v1 == bminv) & (i1 < bmini))
        bmini = jnp.where(t1, i1, bmini)
    cp0 = pltpu.make_async_copy(feats.at[pl.ds(bmaxi, 1), :],
                                out.at[pl.ds(0, 1), :], sem0)
    cp1 = pltpu.make_async_copy(feats.at[pl.ds(bmini, 1), :],
                                out.at[pl.ds(1, 1), :], sem1)
    cp0.start()
    cp1.start()
    cp0.wait()
    cp1.wait()


if N_TC > 0:
    _merge = pl.pallas_call(
        _merge_body,
        in_specs=[
            pl.BlockSpec(memory_space=pltpu.SMEM),
            pl.BlockSpec(memory_space=pltpu.SMEM),
            pl.BlockSpec(memory_space=pltpu.SMEM),
            pl.BlockSpec(memory_space=pltpu.SMEM),
            pl.BlockSpec(memory_space=pl.ANY),
        ],
        out_specs=pl.BlockSpec(memory_space=pltpu.VMEM),
        out_shape=jax.ShapeDtypeStruct((2, D), jnp.float32),
        scratch_shapes=[pltpu.SemaphoreType.DMA, pltpu.SemaphoreType.DMA],
    )
else:
    _merge = pl.pallas_call(
        lambda cand_v, cand_i, feats, out, sem0, sem1: _merge_body(
            cand_v, cand_i, None, None, feats, out, sem0, sem1),
        in_specs=[
            pl.BlockSpec(memory_space=pltpu.SMEM),
            pl.BlockSpec(memory_space=pltpu.SMEM),
            pl.BlockSpec(memory_space=pl.ANY),
        ],
        out_specs=pl.BlockSpec(memory_space=pltpu.VMEM),
        out_shape=jax.ShapeDtypeStruct((2, D), jnp.float32),
        scratch_shapes=[pltpu.SemaphoreType.DMA, pltpu.SemaphoreType.DMA],
    )


def kernel(feats, W, b):
    del b
    vals, idxs = _scan_kernel(feats, W)
    if N_TC > 0:
        wdt = (W[0] - W[1]).reshape(4, 128)
        tcv, tci = _tc_scan(feats, wdt)
        return _merge(vals, idxs, tcv, tci, feats)
    return _merge(vals, idxs, feats)


# speedup vs baseline: 2.1573x; 2.1573x over previous
import functools

import jax
import jax.numpy as jnp
from jax import lax
from jax.experimental import pallas as pl
from jax.experimental.pallas import tpu as pltpu
from jax.experimental.pallas import tpu_sc as plsc

N = 32768
D = 512
LANES = 16
NC = 2
NS = 16
NW = NC * NS

N_SC = N
N_TC = N - N_SC

RPW = N_SC // NW
CH = 32
NBUF = 4
NCHUNK = RPW // CH
KV = D // LANES
ROWU = 2

BR = 256
NB = N_TC // BR

_mesh = plsc.VectorSubcoreMesh(core_axis_name="c", subcore_axis_name="s")


@functools.partial(
    pl.kernel,
    out_type=(
        jax.ShapeDtypeStruct((NW, LANES), jnp.float32),
        jax.ShapeDtypeStruct((NW, LANES), jnp.int32),
    ),
    mesh=_mesh,
    compiler_params=pltpu.CompilerParams(needs_layout_passes=False),
    scratch_types=(
        pltpu.VMEM((2, D), jnp.float32),
        pltpu.VMEM((D,), jnp.float32),
        pltpu.VMEM((NBUF, CH, D), jnp.float32),
        pltpu.VMEM((1, LANES), jnp.float32),
        pltpu.VMEM((1, LANES), jnp.int32),
        (pltpu.SemaphoreType.DMA,) * NBUF,
    ),
)
def _scan_kernel(feats_hbm, w_hbm, vals_out, idx_out, w_v, wd_v, buf, rec_v,
                 rec_i, sems):
    ci = lax.axis_index("c")
    si = lax.axis_index("s")
    wid = si * NC + ci
    base = wid * RPW

    pltpu.sync_copy(w_hbm, w_v)
    for k in range(KV):
        wd_v[pl.ds(LANES * k, LANES)] = (
            w_v[0, pl.ds(LANES * k, LANES)]
            - w_v[1, pl.ds(LANES * k, LANES)])

    def start(cbase, slot):
        return pltpu.async_copy(
            feats_hbm.at[pl.ds(cbase, CH), :], buf.at[slot], sems[slot])

    for p in range(NBUF):
        start(base + p * CH, p)

    wv = [wd_v[pl.ds(LANES * k, LANES)] for k in range(KV)]

    carry = (jnp.float32(-jnp.inf), jnp.int32(0),
             jnp.float32(jnp.inf), jnp.int32(0))

    def make_row_body(slot, cbase):
        def row_body(g, cr):
            r0 = g * ROWU
            dvals = []
            for u in range(ROWU):
                r = r0 + u
                a0 = buf[slot, r, pl.ds(0, LANES)] * wv[0]
                a1 = buf[slot, r, pl.ds(LANES, LANES)] * wv[1]
                for k in range(2, KV, 2):
                    a0 = a0 + buf[slot, r, pl.ds(LANES * k, LANES)] * wv[k]
                    a1 = a1 + (
                        buf[slot, r, pl.ds(LANES * (k + 1), LANES)]
                        * wv[k + 1])
                dvals.append(jnp.sum(a0 + a1))
            bmaxv, bmaxi, bminv, bmini = cr
            for u in range(ROWU):
                d = dvals[u]
                ridx = (cbase + r0 + u).astype(jnp.int32)
                upmax = d > bmaxv
                bmaxv = jnp.where(upmax, d, bmaxv)
                bmaxi = jnp.where(upmax, ridx, bmaxi)
                upmin = d < bminv
                bminv = jnp.where(upmin, d, bminv)
                bmini = jnp.where(upmin, ridx, bmini)
            return (bmaxv, bmaxi, bminv, bmini)

        return row_body

    def chunk_body(g, cr):
        for b in range(NBUF):
            c = NBUF * g + b
            cbase = base + c * CH
            pltpu.make_async_copy(
                feats_hbm.at[pl.ds(cbase, CH), :], buf.at[b],
                sems[b]).wait()
            cr = lax.fori_loop(0, CH // ROWU, make_row_body(b, cbase), cr)

            @pl.when(c + NBUF < NCHUNK)
            def _():
                start(cbase + NBUF * CH, b)
        return cr

    carry = lax.fori_loop(0, NCHUNK // NBUF, chunk_body, carry)
    bmaxv, bmaxi, bminv, bmini = carry
    lane = lax.iota(jnp.int32, LANES)
    rec_v[0] = jnp.where(lane == 0, bmaxv,
                         jnp.where(lane == 1, bminv,
                                   jnp.zeros((LANES,), jnp.float32)))
    rec_i[0] = jnp.where(lane == 0, bmaxi,
                         jnp.where(lane == 1, bmini,
                                   jnp.zeros((LANES,), jnp.int32)))
    pltpu.sync_copy(rec_v, vals_out.at[pl.ds(wid, 1)])
    pltpu.sync_copy(rec_i, idx_out.at[pl.ds(wid, 1)])


def _tc_scan_body(feats_blk, wdt_ref, vals_out, idx_out,
                  bmaxv, bmaxi, bminv, bmini):
    i = pl.program_id(0)

    @pl.when(i == 0)
    def _init():
        bmaxv[...] = jnp.full((BR, 1), -jnp.inf, jnp.float32)
        bmaxi[...] = jnp.zeros((BR, 1), jnp.int32)
        bminv[...] = jnp.full((BR, 1), jnp.inf, jnp.float32)
        bmini[...] = jnp.zeros((BR, 1), jnp.int32)

    x = feats_blk[...].reshape(BR, 4, 128)
    p = x * wdt_ref[...][None, :, :]
    d = jnp.sum(jnp.sum(p, axis=1), axis=1, keepdims=True)
    rows = (N_SC + i * BR
            + lax.broadcasted_iota(jnp.int32, (BR, 1), 0))
    upmax = d > bmaxv[...]
    bmaxv[...] = jnp.where(upmax, d, bmaxv[...])
    bmaxi[...] = jnp.where(upmax, rows, bmaxi[...])
    upmin = d < bminv[...]
    bminv[...] = jnp.where(upmin, d, bminv[...])
    bmini[...] = jnp.where(upmin, rows, bmini[...])

    @pl.when(i == NB - 1)
    def _finish():
        big = jnp.int32(2 ** 30)
        mv = jnp.max(bmaxv[...])
        mi = jnp.min(jnp.where(bmaxv[...] == mv, bmaxi[...], big))
        nv = jnp.min(bminv[...])
        ni = jnp.min(jnp.where(bminv[...] == nv, bmini[...], big))
        col = lax.broadcasted_iota(jnp.int32, (8, 128), 1)
        row8 = lax.broadcasted_iota(jnp.int32, (8, 128), 0)
        first = (col == 0) & (row8 == 0)
        second = (col == 1) & (row8 == 0)
        vals_out[...] = jnp.where(first, mv,
                                  jnp.where(second, nv,
                                            jnp.zeros((8, 128),
                                                      jnp.float32)))
        idx_out[...] = jnp.where(first, mi,
                                 jnp.where(second, ni,
                                           jnp.zeros((8, 128), jnp.int32)))


_tc_scan = None if N_TC == 0 else pl.pallas_call(
    _tc_scan_body,
    grid=(NB,),
    in_specs=[
        pl.BlockSpec((BR, D), lambda i: (i + N_SC // BR, 0)),
        pl.BlockSpec((4, 128), lambda i: (0, 0)),
    ],
    out_specs=[
        pl.BlockSpec((8, 128), lambda i: (0, 0)),
        pl.BlockSpec((8, 128), lambda i: (0, 0)),
    ],
    out_shape=[
        jax.ShapeDtypeStruct((8, 128), jnp.float32),
        jax.ShapeDtypeStruct((8, 128), jnp.int32),
    ],
    scratch_shapes=[
        pltpu.VMEM((BR, 1), jnp.float32),
        pltpu.VMEM((BR, 1), jnp.int32),
        pltpu.VMEM((BR, 1), jnp.float32),
        pltpu.VMEM((BR, 1), jnp.int32),
    ],
)


def _merge_body(cand_v, cand_i, tcv, tci, feats, out, sem0, sem1):
    bmaxv = cand_v[0, 0]
    bmaxi = cand_i[0, 0]
    bminv = cand_v[0, 1]
    bmini = cand_i[0, 1]
    for w in range(1, NW):
        v0 = cand_v[w, 0]
        i0 = cand_i[w, 0]
        t0 = (v0 > bmaxv) | ((v0 == bmaxv) & (i0 < bmaxi))
        bmaxv = jnp.where(t0, v0, bmaxv)
        bmaxi = jnp.where(t0, i0, bmaxi)
        v1 = cand_v[w, 1]
        i1 = cand_i[w, 1]
        t1 = (v1 < bminv) | ((v1 == bminv) & (i1 < bmini))
        bminv = jnp.where(t1, v1, bminv)
        bmini = jnp.where(t1, i1, bmini)
    if tcv is not None:
        v0 = tcv[0, 0]
        i0 = tci[0, 0]
        t0 = (v0 > bmaxv) | ((v0 == bmaxv) & (i0 < bmaxi))
        bmaxi = jnp.where(t0, i0, bmaxi)
        v1 = tcv[0, 1]
        i1 = tci[0, 1]
        t1 = (v1 < bminv) | ((v1 == bminv) & (i1 < bmini))
        bmini = jnp.where(t1, i1, bmini)
    cp0 = pltpu.make_async_copy(feats.at[pl.ds(bmaxi, 1), :],
                                out.at[pl.ds(0, 1), :], sem0)
    cp1 = pltpu.make_async_copy(feats.at[pl.ds(bmini, 1), :],
                                out.at[pl.ds(1, 1), :], sem1)
    cp0.start()
    cp1.start()
    cp0.wait()
    cp1.wait()


if N_TC > 0:
    _merge = pl.pallas_call(
        _merge_body,
        in_specs=[
            pl.BlockSpec(memory_space=pltpu.SMEM),
            pl.BlockSpec(memory_space=pltpu.SMEM),
            pl.BlockSpec(memory_space=pltpu.SMEM),
            pl.BlockSpec(memory_space=pltpu.SMEM),
            pl.BlockSpec(memory_space=pl.ANY),
        ],
        out_specs=pl.BlockSpec(memory_space=pltpu.VMEM),
        out_shape=jax.ShapeDtypeStruct((2, D), jnp.float32),
        scratch_shapes=[pltpu.SemaphoreType.DMA, pltpu.SemaphoreType.DMA],
    )
else:
    _merge = pl.pallas_call(
        lambda cand_v, cand_i, feats, out, sem0, sem1: _merge_body(
            cand_v, cand_i, None, None, feats, out, sem0, sem1),
        in_specs=[
            pl.BlockSpec(memory_space=pltpu.SMEM),
            pl.BlockSpec(memory_space=pltpu.SMEM),
            pl.BlockSpec(memory_space=pl.ANY),
        ],
        out_specs=pl.BlockSpec(memory_space=pltpu.VMEM),
        out_shape=jax.ShapeDtypeStruct((2, D), jnp.float32),
        scratch_shapes=[pltpu.SemaphoreType.DMA, pltpu.SemaphoreType.DMA],
    )


def kernel(feats, W, b):
    del b
    vals, idxs = _scan_kernel(feats, W)
    if N_TC > 0:
        wdt = (W[0] - W[1]).reshape(4, 128)
        tcv, tci = _tc_scan(feats, wdt)
        return _merge(vals, idxs, tcv, tci, feats)
    return _merge(vals, idxs, feats)


# speedup vs baseline: 2.2402x; 1.0384x over previous
import functools

import jax
import jax.numpy as jnp
from jax import lax
from jax.experimental import pallas as pl
from jax.experimental.pallas import tpu as pltpu
from jax.experimental.pallas import tpu_sc as plsc

N = 32768
D = 512
LANES = 16
NC = 2
NS = 16
NW = NC * NS

N_SC = N
N_TC = N - N_SC

RPW = N_SC // NW
CH = 32
NBUF = 4
NCHUNK = RPW // CH
KV = D // LANES
ROWU = 2

BR = 256
NB = N_TC // BR

_mesh = plsc.VectorSubcoreMesh(core_axis_name="c", subcore_axis_name="s")


@functools.partial(
    pl.kernel,
    out_type=(
        jax.ShapeDtypeStruct((NW, LANES), jnp.float32),
        jax.ShapeDtypeStruct((NW, LANES), jnp.int32),
    ),
    mesh=_mesh,
    compiler_params=pltpu.CompilerParams(needs_layout_passes=False),
    scratch_types=(
        pltpu.VMEM((2, D), jnp.float32),
        pltpu.VMEM((D,), jnp.float32),
        pltpu.VMEM((NBUF, CH, D), jnp.float32),
        pltpu.VMEM((1, LANES), jnp.float32),
        pltpu.VMEM((1, LANES), jnp.int32),
        (pltpu.SemaphoreType.DMA,) * NBUF,
    ),
)
def _scan_kernel(feats_hbm, w_hbm, vals_out, idx_out, w_v, wd_v, buf, rec_v,
                 rec_i, sems):
    ci = lax.axis_index("c")
    si = lax.axis_index("s")
    wid = si * NC + ci
    base = wid * RPW

    pltpu.sync_copy(w_hbm, w_v)
    for k in range(KV):
        wd_v[pl.ds(LANES * k, LANES)] = (
            w_v[0, pl.ds(LANES * k, LANES)]
            - w_v[1, pl.ds(LANES * k, LANES)])

    def start(cbase, slot):
        return pltpu.async_copy(
            feats_hbm.at[pl.ds(cbase, CH), :], buf.at[slot], sems[slot])

    for p in range(NBUF):
        start(base + p * CH, p)

    wv = [wd_v[pl.ds(LANES * k, LANES)] for k in range(KV)]

    carry = (jnp.float32(-jnp.inf), jnp.int32(0),
             jnp.float32(jnp.inf), jnp.int32(0))

    def make_row_body(slot, cbase):
        def row_body(g, cr):
            r0 = g * ROWU
            dvals = []
            for u in range(ROWU):
                r = r0 + u
                acc = [buf[slot, r, pl.ds(LANES * k, LANES)] * wv[k]
                       for k in range(4)]
                for k in range(4, KV):
                    acc[k % 4] = acc[k % 4] + (
                        buf[slot, r, pl.ds(LANES * k, LANES)] * wv[k])
                dvals.append(jnp.sum((acc[0] + acc[1]) + (acc[2] + acc[3])))
            bmaxv, bmaxi, bminv, bmini = cr
            for u in range(ROWU):
                d = dvals[u]
                ridx = (cbase + r0 + u).astype(jnp.int32)
                upmax = d > bmaxv
                bmaxv = jnp.where(upmax, d, bmaxv)
                bmaxi = jnp.where(upmax, ridx, bmaxi)
                upmin = d < bminv
                bminv = jnp.where(upmin, d, bminv)
                bmini = jnp.where(upmin, ridx, bmini)
            return (bmaxv, bmaxi, bminv, bmini)

        return row_body

    def chunk_body(g, cr):
        for b in range(NBUF):
            c = NBUF * g + b
            cbase = base + c * CH
            pltpu.make_async_copy(
                feats_hbm.at[pl.ds(cbase, CH), :], buf.at[b],
                sems[b]).wait()
            cr = lax.fori_loop(0, CH // ROWU, make_row_body(b, cbase), cr)

            @pl.when(c + NBUF < NCHUNK)
            def _():
                start(cbase + NBUF * CH, b)
        return cr

    carry = lax.fori_loop(0, NCHUNK // NBUF, chunk_body, carry)
    bmaxv, bmaxi, bminv, bmini = carry
    lane = lax.iota(jnp.int32, LANES)
    rec_v[0] = jnp.where(lane == 0, bmaxv,
                         jnp.where(lane == 1, bminv,
                                   jnp.zeros((LANES,), jnp.float32)))
    rec_i[0] = jnp.where(lane == 0, bmaxi,
                         jnp.where(lane == 1, bmini,
                                   jnp.zeros((LANES,), jnp.int32)))
    pltpu.sync_copy(rec_v, vals_out.at[pl.ds(wid, 1)])
    pltpu.sync_copy(rec_i, idx_out.at[pl.ds(wid, 1)])


def _tc_scan_body(feats_blk, wdt_ref, vals_out, idx_out,
                  bmaxv, bmaxi, bminv, bmini):
    i = pl.program_id(0)

    @pl.when(i == 0)
    def _init():
        bmaxv[...] = jnp.full((BR, 1), -jnp.inf, jnp.float32)
        bmaxi[...] = jnp.zeros((BR, 1), jnp.int32)
        bminv[...] = jnp.full((BR, 1), jnp.inf, jnp.float32)
        bmini[...] = jnp.zeros((BR, 1), jnp.int32)

    x = feats_blk[...].reshape(BR, 4, 128)
    p = x * wdt_ref[...][None, :, :]
    d = jnp.sum(jnp.sum(p, axis=1), axis=1, keepdims=True)
    rows = (N_SC + i * BR
            + lax.broadcasted_iota(jnp.int32, (BR, 1), 0))
    upmax = d > bmaxv[...]
    bmaxv[...] = jnp.where(upmax, d, bmaxv[...])
    bmaxi[...] = jnp.where(upmax, rows, bmaxi[...])
    upmin = d < bminv[...]
    bminv[...] = jnp.where(upmin, d, bminv[...])
    bmini[...] = jnp.where(upmin, rows, bmini[...])

    @pl.when(i == NB - 1)
    def _finish():
        big = jnp.int32(2 ** 30)
        mv = jnp.max(bmaxv[...])
        mi = jnp.min(jnp.where(bmaxv[...] == mv, bmaxi[...], big))
        nv = jnp.min(bminv[...])
        ni = jnp.min(jnp.where(bminv[...] == nv, bmini[...], big))
        col = lax.broadcasted_iota(jnp.int32, (8, 128), 1)
        row8 = lax.broadcasted_iota(jnp.int32, (8, 128), 0)
        first = (col == 0) & (row8 == 0)
        second = (col == 1) & (row8 == 0)
        vals_out[...] = jnp.where(first, mv,
                                  jnp.where(second, nv,
                                            jnp.zeros((8, 128),
                                                      jnp.float32)))
        idx_out[...] = jnp.where(first, mi,
                                 jnp.where(second, ni,
                                           jnp.zeros((8, 128), jnp.int32)))


_tc_scan = None if N_TC == 0 else pl.pallas_call(
    _tc_scan_body,
    grid=(NB,),
    in_specs=[
        pl.BlockSpec((BR, D), lambda i: (i + N_SC // BR, 0)),
        pl.BlockSpec((4, 128), lambda i: (0, 0)),
    ],
    out_specs=[
        pl.BlockSpec((8, 128), lambda i: (0, 0)),
        pl.BlockSpec((8, 128), lambda i: (0, 0)),
    ],
    out_shape=[
        jax.ShapeDtypeStruct((8, 128), jnp.float32),
        jax.ShapeDtypeStruct((8, 128), jnp.int32),
    ],
    scratch_shapes=[
        pltpu.VMEM((BR, 1), jnp.float32),
        pltpu.VMEM((BR, 1), jnp.int32),
        pltpu.VMEM((BR, 1), jnp.float32),
        pltpu.VMEM((BR, 1), jnp.int32),
    ],
)


def _merge_body(cand_v, cand_i, tcv, tci, feats, out, sem0, sem1):
    bmaxv = cand_v[0, 0]
    bmaxi = cand_i[0, 0]
    bminv = cand_v[0, 1]
    bmini = cand_i[0, 1]
    for w in range(1, NW):
        v0 = cand_v[w, 0]
        i0 = cand_i[w, 0]
        t0 = (v0 > bmaxv) | ((v0 == bmaxv) & (i0 < bmaxi))
        bmaxv = jnp.where(t0, v0, bmaxv)
        bmaxi = jnp.where(t0, i0, bmaxi)
        v1 = cand_v[w, 1]
        i1 = cand_i[w, 1]
        t1 = (v1 < bminv) | ((v1 == bminv) & (i1 < bmini))
        bminv = jnp.where(t1, v1, bminv)
        bmini = jnp.where(t1, i1, bmini)
    if tcv is not None:
        v0 = tcv[0, 0]
        i0 = tci[0, 0]
        t0 = (v0 > bmaxv) | ((v0 == bmaxv) & (i0 < bmaxi))
        bmaxi = jnp.where(t0, i0, bmaxi)
        v1 = tcv[0, 1]
        i1 = tci[0, 1]
        t1 = (v1 < bminv) | ((v1 == bminv) & (i1 < bmini))
        bmini = jnp.where(t1, i1, bmini)
    cp0 = pltpu.make_async_copy(feats.at[pl.ds(bmaxi, 1), :],
                                out.at[pl.ds(0, 1), :], sem0)
    cp1 = pltpu.make_async_copy(feats.at[pl.ds(bmini, 1), :],
                                out.at[pl.ds(1, 1), :], sem1)
    cp0.start()
    cp1.start()
    cp0.wait()
    cp1.wait()


if N_TC > 0:
    _merge = pl.pallas_call(
        _merge_body,
        in_specs=[
            pl.BlockSpec(memory_space=pltpu.SMEM),
            pl.BlockSpec(memory_space=pltpu.SMEM),
            pl.BlockSpec(memory_space=pltpu.SMEM),
            pl.BlockSpec(memory_space=pltpu.SMEM),
            pl.BlockSpec(memory_space=pl.ANY),
        ],
        out_specs=pl.BlockSpec(memory_space=pltpu.VMEM),
        out_shape=jax.ShapeDtypeStruct((2, D), jnp.float32),
        scratch_shapes=[pltpu.SemaphoreType.DMA, pltpu.SemaphoreType.DMA],
    )
else:
    _merge = pl.pallas_call(
        lambda cand_v, cand_i, feats, out, sem0, sem1: _merge_body(
            cand_v, cand_i, None, None, feats, out, sem0, sem1),
        in_specs=[
            pl.BlockSpec(memory_space=pltpu.SMEM),
            pl.BlockSpec(memory_space=pltpu.SMEM),
            pl.BlockSpec(memory_space=pl.ANY),
        ],
        out_specs=pl.BlockSpec(memory_space=pltpu.VMEM),
        out_shape=jax.ShapeDtypeStruct((2, D), jnp.float32),
        scratch_shapes=[pltpu.SemaphoreType.DMA, pltpu.SemaphoreType.DMA],
    )


def kernel(feats, W, b):
    del b
    vals, idxs = _scan_kernel(feats, W)
    if N_TC > 0:
        wdt = (W[0] - W[1]).reshape(4, 128)
        tcv, tci = _tc_scan(feats, wdt)
        return _merge(vals, idxs, tcv, tci, feats)
    return _merge(vals, idxs, feats)
